# Initial kernel scaffold; baseline (speedup 1.0000x reference)
import jax
import jax.numpy as jnp
from jax.experimental import pallas as pl


def kernel(x, edge_index, label, train_idx):
    raise NotImplementedError("write your pallas kernel here")


# speedup vs baseline: 9.1132x; 9.1132x over previous
import functools

import jax
import jax.numpy as jnp
from jax import lax
from jax.experimental import pallas as pl
from jax.experimental.pallas import tpu as pltpu
from jax.experimental.pallas import tpu_sc as plsc

N = 10000
E = 320000
C = 40
ALPHA = 0.9
HOPS = 2
NUM_ITERS = 50

NS = 16
L = 16
CP = 48
NP = 10240
RT = NP // NS
RB = 128
NRC = RT // RB
EB = 128
NB = 162
EPAD = NS * NB * EB


def _lp_body(colt, rowt, d2e, ybe, dinve, out, s_hbm, a_sh,
             colv, rowv, gbuf, da, dd, dy):
  sid = lax.axis_index("s")
  cid = lax.axis_index("c")
  base_row = sid * RT

  def dense(mode):
    @pl.loop(0, NRC)
    def _chunk(k):
      rb = base_row + k * RB
      fb = rb * CP
      if mode != 0:
        pltpu.sync_copy(a_sh.at[pl.ds(rb, RB)], da)
        pltpu.sync_copy(d2e.at[pl.ds(fb, RB * CP)], dd)
      if mode != 1:
        pltpu.sync_copy(ybe.at[pl.ds(fb, RB * CP)], dy)

      @pl.loop(0, RB)
      def _row(r):
        for j in range(CP // L):
          sl = pl.ds(j * L, L)
          fsl = pl.ds(r * CP + j * L, L)
          if mode == 0:
            da[r, sl] = dy[fsl] * 10.0
          elif mode == 1:
            da[r, sl] = da[r, sl] * dd[fsl]
          else:
            da[r, sl] = da[r, sl] * dd[fsl] * ALPHA + dy[fsl]

      pltpu.sync_copy(da, s_hbm.at[pl.ds(rb, RB)])

      @pl.loop(0, RB)
      def _zrow(r):
        for j in range(CP // L):
          da[r, pl.ds(j * L, L)] = jnp.zeros((L,), jnp.float32)

      pltpu.sync_copy(da, a_sh.at[pl.ds(rb, RB)])

  def edge_pass():
    @pl.loop(0, NB)
    def _blk(b):
      pltpu.sync_copy(s_hbm.at[colv.at[b]], gbuf)
      pltpu.sync_copy(gbuf, a_sh.at[rowv.at[b]], add=True)

  @pl.when(cid == 0)
  def _run():
    pltpu.sync_copy(colt.at[sid], colv)
    pltpu.sync_copy(rowt.at[sid], rowv)

    dense(0)
    plsc.subcore_barrier()

    @pl.loop(0, NUM_ITERS)
    def _iter(i):
      edge_pass()
      plsc.subcore_barrier()
      dense(1)
      plsc.subcore_barrier()
      edge_pass()
      plsc.subcore_barrier()
      dense(2)
      plsc.subcore_barrier()

    @pl.loop(0, NRC)
    def _chunk(k):
      rb = base_row + k * RB
      pltpu.sync_copy(s_hbm.at[pl.ds(rb, RB)], da)
      pltpu.sync_copy(dinve.at[pl.ds(rb * CP, RB * CP)], dd)

      @pl.loop(0, RB)
      def _row(r):
        for j in range(CP // L):
          sl = pl.ds(j * L, L)
          da[r, sl] = da[r, sl] * dd[pl.ds(r * CP + j * L, L)]

      pltpu.sync_copy(da, out.at[pl.ds(rb, RB)])


_lp_call = functools.partial(
    pl.kernel,
    out_type=(jax.ShapeDtypeStruct((NP, CP), jnp.float32),
              jax.ShapeDtypeStruct((NP, CP), jnp.float32)),
    mesh=plsc.VectorSubcoreMesh(core_axis_name="c", subcore_axis_name="s"),
    compiler_params=pltpu.CompilerParams(use_tc_tiling_on_sc=False),
    scratch_types=[
        pltpu.VMEM_SHARED((NP, CP), jnp.float32),
        pltpu.VMEM((NB, EB), jnp.int32),
        pltpu.VMEM((NB, EB), jnp.int32),
        pltpu.VMEM((EB, CP), jnp.float32),
        pltpu.VMEM((RB, CP), jnp.float32),
        pltpu.VMEM((RB * CP,), jnp.float32),
        pltpu.VMEM((RB * CP,), jnp.float32),
    ],
)(_lp_body)


def kernel(x, edge_index, label, train_idx):
  n = x.shape[0]
  row = edge_index[0]
  col = edge_index[1]
  loop = jnp.arange(n, dtype=row.dtype)
  rowf = jnp.concatenate([row, loop])
  colf = jnp.concatenate([col, loop])

  deg = jnp.zeros((n,), jnp.float32).at[colf].add(1.0)
  dis = lax.rsqrt(deg)
  d2 = dis * dis

  oh = jax.nn.one_hot(label[train_idx, 0], C, dtype=jnp.float32)
  y = jnp.zeros((n, C), jnp.float32).at[train_idx].set(oh)
  yb = 0.1 * dis[:, None] * y

  def expand(v):
    if v.ndim == 1:
      v = jnp.broadcast_to(v[:, None], (n, C))
    vp = jnp.zeros((NP, CP), jnp.float32).at[:n, :C].set(v)
    return vp.reshape(-1)

  d2e = expand(d2)
  ybe = expand(yb)
  dinve = expand(jnp.sqrt(deg))

  pad = jnp.full((EPAD - (E + n),), NP - 1, dtype=jnp.int32)
  colt = jnp.concatenate([colf, pad]).reshape(NS, NB, EB)
  rowt = jnp.concatenate([rowf, pad]).reshape(NS, NB, EB)

  out, _ = _lp_call(colt, rowt, d2e, ybe, dinve)
  return out[:n, :C]


# speedup vs baseline: 17.4511x; 1.9149x over previous
import functools

import jax
import jax.numpy as jnp
from jax import lax
from jax.experimental import pallas as pl
from jax.experimental.pallas import tpu as pltpu
from jax.experimental.pallas import tpu_sc as plsc

N = 10000
E = 320000
C = 40
ALPHA = 0.9
HOPS = 2
NUM_ITERS = 50

NS = 16
L = 16
CP = 48
NP = 10240
RT = NP // NS
RB = 128
NRC = RT // RB
EB = 128
NB = 162
EPAD = NS * NB * EB


NBUF = 3


def _lp_body(colt, rowt, d2e, ybe, dinve, out, s_hbm, a_sh,
             colv, rowv, gb0, gb1, gb2, da, dd, dy,
             gsem0, gsem1, gsem2, ssem0, ssem1, ssem2):
  gb = (gb0, gb1, gb2)
  gsem = (gsem0, gsem1, gsem2)
  ssem = (ssem0, ssem1, ssem2)
  sid = lax.axis_index("s")
  cid = lax.axis_index("c")
  base_row = sid * RT

  def dense(mode):
    @pl.loop(0, NRC)
    def _chunk(k):
      rb = base_row + k * RB
      fb = rb * CP
      if mode != 0:
        pltpu.sync_copy(a_sh.at[pl.ds(rb, RB)], da)
        pltpu.sync_copy(d2e.at[pl.ds(fb, RB * CP)], dd)
      if mode != 1:
        pltpu.sync_copy(ybe.at[pl.ds(fb, RB * CP)], dy)

      @pl.loop(0, RB)
      def _row(r):
        for j in range(CP // L):
          sl = pl.ds(j * L, L)
          fsl = pl.ds(r * CP + j * L, L)
          if mode == 0:
            da[r, sl] = dy[fsl] * 10.0
          elif mode == 1:
            da[r, sl] = da[r, sl] * dd[fsl]
          else:
            da[r, sl] = da[r, sl] * dd[fsl] * ALPHA + dy[fsl]

      pltpu.sync_copy(da, s_hbm.at[pl.ds(rb, RB)])

      @pl.loop(0, RB)
      def _zrow(r):
        for j in range(CP // L):
          da[r, pl.ds(j * L, L)] = jnp.zeros((L,), jnp.float32)

      pltpu.sync_copy(da, a_sh.at[pl.ds(rb, RB)])

  def edge_pass():
    def fire_g(b, k):
      pltpu.async_copy(s_hbm.at[colv.at[b]], gb[k], gsem[k])

    def wait_g(k):
      pltpu.make_async_copy(s_hbm.at[colv.at[0]], gb[k], gsem[k]).wait()

    def fire_s(b, k):
      pltpu.async_copy(gb[k], a_sh.at[rowv.at[b]], ssem[k], add=True)

    def wait_s(k):
      pltpu.make_async_copy(gb[k], a_sh.at[rowv.at[0]], ssem[k]).wait()

    fire_g(0, 0)
    fire_g(1, 1)
    wait_g(0)
    fire_s(0, 0)
    fire_g(2, 2)
    wait_g(1)
    fire_s(1, 1)

    @pl.loop(0, (NB - NBUF) // NBUF)
    def _grp(i):
      b0 = 2 + i * NBUF
      for j in range(NBUF):
        b = b0 + j
        k = (2 + j) % NBUF
        kn = (k + 1) % NBUF
        wait_s(kn)
        fire_g(b + 1, kn)
        wait_g(k)
        fire_s(b, k)

    wait_g((NB - 1) % NBUF)
    fire_s(NB - 1, (NB - 1) % NBUF)
    for k in range(NBUF):
      wait_s(k)

  @pl.when(cid == 0)
  def _run():
    pltpu.sync_copy(colt.at[sid], colv)
    pltpu.sync_copy(rowt.at[sid], rowv)

    dense(0)
    plsc.subcore_barrier()

    @pl.loop(0, NUM_ITERS)
    def _iter(i):
      edge_pass()
      plsc.subcore_barrier()
      dense(1)
      plsc.subcore_barrier()
      edge_pass()
      plsc.subcore_barrier()
      dense(2)
      plsc.subcore_barrier()

    @pl.loop(0, NRC)
    def _chunk(k):
      rb = base_row + k * RB
      pltpu.sync_copy(s_hbm.at[pl.ds(rb, RB)], da)
      pltpu.sync_copy(dinve.at[pl.ds(rb * CP, RB * CP)], dd)

      @pl.loop(0, RB)
      def _row(r):
        for j in range(CP // L):
          sl = pl.ds(j * L, L)
          da[r, sl] = da[r, sl] * dd[pl.ds(r * CP + j * L, L)]

      pltpu.sync_copy(da, out.at[pl.ds(rb, RB)])


_lp_call = functools.partial(
    pl.kernel,
    out_type=(jax.ShapeDtypeStruct((NP, CP), jnp.float32),
              jax.ShapeDtypeStruct((NP, CP), jnp.float32)),
    mesh=plsc.VectorSubcoreMesh(core_axis_name="c", subcore_axis_name="s"),
    compiler_params=pltpu.CompilerParams(use_tc_tiling_on_sc=False),
    scratch_types=[
        pltpu.VMEM_SHARED((NP, CP), jnp.float32),
        pltpu.VMEM((NB, EB), jnp.int32),
        pltpu.VMEM((NB, EB), jnp.int32),
        pltpu.VMEM((EB, CP), jnp.float32),
        pltpu.VMEM((EB, CP), jnp.float32),
        pltpu.VMEM((EB, CP), jnp.float32),
        pltpu.VMEM((RB, CP), jnp.float32),
        pltpu.VMEM((RB * CP,), jnp.float32),
        pltpu.VMEM((RB * CP,), jnp.float32),
        pltpu.SemaphoreType.DMA,
        pltpu.SemaphoreType.DMA,
        pltpu.SemaphoreType.DMA,
        pltpu.SemaphoreType.DMA,
        pltpu.SemaphoreType.DMA,
        pltpu.SemaphoreType.DMA,
    ],
)(_lp_body)


def kernel(x, edge_index, label, train_idx):
  n = x.shape[0]
  row = edge_index[0]
  col = edge_index[1]
  loop = jnp.arange(n, dtype=row.dtype)
  rowf = jnp.concatenate([row, loop])
  colf = jnp.concatenate([col, loop])

  deg = jnp.zeros((n,), jnp.float32).at[colf].add(1.0)
  dis = lax.rsqrt(deg)
  d2 = dis * dis

  oh = jax.nn.one_hot(label[train_idx, 0], C, dtype=jnp.float32)
  y = jnp.zeros((n, C), jnp.float32).at[train_idx].set(oh)
  yb = 0.1 * dis[:, None] * y

  def expand(v):
    if v.ndim == 1:
      v = jnp.broadcast_to(v[:, None], (n, C))
    vp = jnp.zeros((NP, CP), jnp.float32).at[:n, :C].set(v)
    return vp.reshape(-1)

  d2e = expand(d2)
  ybe = expand(yb)
  dinve = expand(jnp.sqrt(deg))

  pad = jnp.full((EPAD - (E + n),), NP - 1, dtype=jnp.int32)
  colt = jnp.concatenate([colf, pad]).reshape(NS, NB, EB)
  rowt = jnp.concatenate([rowf, pad]).reshape(NS, NB, EB)

  out, _ = _lp_call(colt, rowt, d2e, ybe, dinve)
  return out[:n, :C]


# speedup vs baseline: 19.8608x; 1.1381x over previous
import functools

import jax
import jax.numpy as jnp
from jax import lax
from jax.experimental import pallas as pl
from jax.experimental.pallas import tpu as pltpu
from jax.experimental.pallas import tpu_sc as plsc

N = 10000
E = 320000
C = 40
ALPHA = 0.9
NUM_ITERS = 50

NS = 16
L = 16
CH = 32
NP = 10240
RT = NP // NS
RB = 128
NRC = RT // RB
EB = 128
NB = 162
EPAD = NS * NB * EB
NBUF = 3


def _lp_body(colt, rowt, d2e, ybe, dinve, out, s_hbm, a_sh,
             colv, rowv, gb0, gb1, gb2, da, dd, dy,
             gsem0, gsem1, gsem2, ssem0, ssem1, ssem2):
  gb = (gb0, gb1, gb2)
  gsem = (gsem0, gsem1, gsem2)
  ssem = (ssem0, ssem1, ssem2)
  sid = lax.axis_index("s")
  cid = lax.axis_index("c")
  base_row = sid * RT

  pltpu.sync_copy(colt.at[sid], colv)
  pltpu.sync_copy(rowt.at[sid], rowv)

  def dense(mode):
    @pl.loop(0, NRC)
    def _chunk(k):
      rb = base_row + k * RB
      if mode != 0:
        pltpu.sync_copy(a_sh.at[pl.ds(rb, RB)], da)
        pltpu.sync_copy(d2e.at[cid, pl.ds(rb, RB)], dd)
      if mode != 1:
        pltpu.sync_copy(ybe.at[cid, pl.ds(rb, RB)], dy)

      @pl.loop(0, RB)
      def _row(r):
        for j in range(CH // L):
          sl = pl.ds(j * L, L)
          if mode == 0:
            da[r, sl] = dy[r, sl] * 10.0
          elif mode == 1:
            da[r, sl] = da[r, sl] * dd[r, sl]
          else:
            da[r, sl] = da[r, sl] * dd[r, sl] * ALPHA + dy[r, sl]

      pltpu.sync_copy(da, s_hbm.at[cid, pl.ds(rb, RB)])

      @pl.loop(0, RB)
      def _zrow(r):
        for j in range(CH // L):
          da[r, pl.ds(j * L, L)] = jnp.zeros((L,), jnp.float32)

      pltpu.sync_copy(da, a_sh.at[pl.ds(rb, RB)])

  def edge_pass():
    def fire_g(b, k):
      pltpu.async_copy(s_hbm.at[cid].at[colv.at[b]], gb[k], gsem[k])

    def wait_g(k):
      pltpu.make_async_copy(s_hbm.at[cid].at[colv.at[0]], gb[k],
                            gsem[k]).wait()

    def fire_s(b, k):
      pltpu.async_copy(gb[k], a_sh.at[rowv.at[b]], ssem[k], add=True)

    def wait_s(k):
      pltpu.make_async_copy(gb[k], a_sh.at[rowv.at[0]], ssem[k]).wait()

    fire_g(0, 0)
    fire_g(1, 1)
    wait_g(0)
    fire_s(0, 0)
    fire_g(2, 2)
    wait_g(1)
    fire_s(1, 1)

    @pl.loop(0, (NB - NBUF) // NBUF)
    def _grp(i):
      b0 = 2 + i * NBUF
      for j in range(NBUF):
        b = b0 + j
        k = (2 + j) % NBUF
        kn = (k + 1) % NBUF
        wait_s(kn)
        fire_g(b + 1, kn)
        wait_g(k)
        fire_s(b, k)

    wait_g((NB - 1) % NBUF)
    fire_s(NB - 1, (NB - 1) % NBUF)
    for k in range(NBUF):
      wait_s(k)

  dense(0)
  plsc.subcore_barrier()

  @pl.loop(0, NUM_ITERS)
  def _iter(i):
    edge_pass()
    plsc.subcore_barrier()
    dense(1)
    plsc.subcore_barrier()
    edge_pass()
    plsc.subcore_barrier()
    dense(2)
    plsc.subcore_barrier()

  @pl.loop(0, NRC)
  def _chunk(k):
    rb = base_row + k * RB
    pltpu.sync_copy(s_hbm.at[cid, pl.ds(rb, RB)], da)
    pltpu.sync_copy(dinve.at[cid, pl.ds(rb, RB)], dd)

    @pl.loop(0, RB)
    def _row(r):
      for j in range(CH // L):
        sl = pl.ds(j * L, L)
        da[r, sl] = da[r, sl] * dd[r, sl]

    pltpu.sync_copy(da, out.at[cid, pl.ds(rb, RB)])


_lp_call = functools.partial(
    pl.kernel,
    out_type=(jax.ShapeDtypeStruct((2, NP, CH), jnp.float32),
              jax.ShapeDtypeStruct((2, NP, CH), jnp.float32)),
    mesh=plsc.VectorSubcoreMesh(core_axis_name="c", subcore_axis_name="s"),
    compiler_params=pltpu.CompilerParams(use_tc_tiling_on_sc=False),
    scratch_types=[
        pltpu.VMEM_SHARED((NP, CH), jnp.float32),
        pltpu.VMEM((NB, EB), jnp.int32),
        pltpu.VMEM((NB, EB), jnp.int32),
        pltpu.VMEM((EB, CH), jnp.float32),
        pltpu.VMEM((EB, CH), jnp.float32),
        pltpu.VMEM((EB, CH), jnp.float32),
        pltpu.VMEM((RB, CH), jnp.float32),
        pltpu.VMEM((RB, CH), jnp.float32),
        pltpu.VMEM((RB, CH), jnp.float32),
        pltpu.SemaphoreType.DMA,
        pltpu.SemaphoreType.DMA,
        pltpu.SemaphoreType.DMA,
        pltpu.SemaphoreType.DMA,
        pltpu.SemaphoreType.DMA,
        pltpu.SemaphoreType.DMA,
    ],
)(_lp_body)


def kernel(x, edge_index, label, train_idx):
  n = x.shape[0]
  row = edge_index[0]
  col = edge_index[1]
  loop = jnp.arange(n, dtype=row.dtype)
  rowf = jnp.concatenate([row, loop])
  colf = jnp.concatenate([col, loop])

  deg = jnp.zeros((n,), jnp.float32).at[colf].add(1.0)
  dis = lax.rsqrt(deg)
  d2 = dis * dis

  oh = jax.nn.one_hot(label[train_idx, 0], C, dtype=jnp.float32)
  y = jnp.zeros((n, C), jnp.float32).at[train_idx].set(oh)
  yb = 0.1 * dis[:, None] * y

  def expand(v):
    if v.ndim == 1:
      v = jnp.broadcast_to(v[:, None], (n, 2 * CH))
    else:
      v = jnp.pad(v, ((0, 0), (0, 2 * CH - v.shape[1])))
    vp = jnp.zeros((NP, 2 * CH), jnp.float32).at[:n].set(v)
    return vp.reshape(NP, 2, CH).transpose(1, 0, 2)

  d2e = expand(d2)
  ybe = expand(yb)
  dinve = expand(jnp.sqrt(deg))

  pad = jnp.full((EPAD - (E + n),), NP - 1, dtype=jnp.int32)
  colt = jnp.concatenate([colf, pad]).reshape(NS, NB, EB)
  rowt = jnp.concatenate([rowf, pad]).reshape(NS, NB, EB)

  out, _ = _lp_call(colt, rowt, d2e, ybe, dinve)
  res = jnp.concatenate([out[0], out[1]], axis=1)
  return res[:n, :C]


# speedup vs baseline: 24.3975x; 1.2284x over previous
import functools

import jax
import jax.numpy as jnp
from jax import lax
from jax.experimental import pallas as pl
from jax.experimental.pallas import tpu as pltpu
from jax.experimental.pallas import tpu_sc as plsc

N = 10000
E = 320000
C = 40
ALPHA = 0.9
NUM_ITERS = 50

NS = 16
L = 16
CH = 32
NP = 10240
RT = NP // NS
RB = 128
NRC = RT // RB
EB = 256
NB = 81
EPAD = NS * NB * EB
NBUF = 3


def _lp_body(colt, rowt, d2e, ybe, dinve, out, s_hbm, a_sh,
             colv, rowv, gb0, gb1, gb2, da, dd, dy,
             gsem0, gsem1, gsem2, ssem0, ssem1, ssem2):
  gb = (gb0, gb1, gb2)
  gsem = (gsem0, gsem1, gsem2)
  ssem = (ssem0, ssem1, ssem2)
  sid = lax.axis_index("s")
  cid = lax.axis_index("c")
  base_row = sid * RT

  pltpu.sync_copy(colt.at[sid], colv)
  pltpu.sync_copy(rowt.at[sid], rowv)

  def dense(mode):
    @pl.loop(0, NRC)
    def _chunk(k):
      rb = base_row + k * RB
      if mode != 0:
        pltpu.sync_copy(a_sh.at[pl.ds(rb, RB)], da)
        pltpu.sync_copy(d2e.at[cid, pl.ds(rb, RB)], dd)
      if mode != 1:
        pltpu.sync_copy(ybe.at[cid, pl.ds(rb, RB)], dy)

      @pl.loop(0, RB)
      def _row(r):
        for j in range(CH // L):
          sl = pl.ds(j * L, L)
          if mode == 0:
            da[r, sl] = dy[r, sl] * 10.0
          elif mode == 1:
            da[r, sl] = da[r, sl] * dd[r, sl]
          else:
            da[r, sl] = da[r, sl] * dd[r, sl] * ALPHA + dy[r, sl]

      pltpu.sync_copy(da, s_hbm.at[cid, pl.ds(rb, RB)])

      @pl.loop(0, RB)
      def _zrow(r):
        for j in range(CH // L):
          da[r, pl.ds(j * L, L)] = jnp.zeros((L,), jnp.float32)

      pltpu.sync_copy(da, a_sh.at[pl.ds(rb, RB)])

  def edge_pass():
    def fire_g(b, k):
      pltpu.async_copy(s_hbm.at[cid].at[colv.at[b]], gb[k], gsem[k])

    def wait_g(k):
      pltpu.make_async_copy(s_hbm.at[cid].at[colv.at[0]], gb[k],
                            gsem[k]).wait()

    def fire_s(b, k):
      pltpu.async_copy(gb[k], a_sh.at[rowv.at[b]], ssem[k], add=True)

    def wait_s(k):
      pltpu.make_async_copy(gb[k], a_sh.at[rowv.at[0]], ssem[k]).wait()

    fire_g(0, 0)
    fire_g(1, 1)
    wait_g(0)
    fire_s(0, 0)
    fire_g(2, 2)
    wait_g(1)
    fire_s(1, 1)

    @pl.loop(0, (NB - NBUF) // NBUF)
    def _grp(i):
      b0 = 2 + i * NBUF
      for j in range(NBUF):
        b = b0 + j
        k = (2 + j) % NBUF
        kn = (k + 1) % NBUF
        wait_s(kn)
        fire_g(b + 1, kn)
        wait_g(k)
        fire_s(b, k)

    wait_g((NB - 1) % NBUF)
    fire_s(NB - 1, (NB - 1) % NBUF)
    for k in range(NBUF):
      wait_s(k)

  dense(0)
  plsc.subcore_barrier()

  @pl.loop(0, NUM_ITERS)
  def _iter(i):
    edge_pass()
    plsc.subcore_barrier()
    dense(1)
    plsc.subcore_barrier()
    edge_pass()
    plsc.subcore_barrier()
    dense(2)
    plsc.subcore_barrier()

  @pl.loop(0, NRC)
  def _chunk(k):
    rb = base_row + k * RB
    pltpu.sync_copy(s_hbm.at[cid, pl.ds(rb, RB)], da)
    pltpu.sync_copy(dinve.at[cid, pl.ds(rb, RB)], dd)

    @pl.loop(0, RB)
    def _row(r):
      for j in range(CH // L):
        sl = pl.ds(j * L, L)
        da[r, sl] = da[r, sl] * dd[r, sl]

    pltpu.sync_copy(da, out.at[cid, pl.ds(rb, RB)])


_lp_call = functools.partial(
    pl.kernel,
    out_type=(jax.ShapeDtypeStruct((2, NP, CH), jnp.float32),
              jax.ShapeDtypeStruct((2, NP, CH), jnp.float32)),
    mesh=plsc.VectorSubcoreMesh(core_axis_name="c", subcore_axis_name="s"),
    compiler_params=pltpu.CompilerParams(use_tc_tiling_on_sc=False),
    scratch_types=[
        pltpu.VMEM_SHARED((NP, CH), jnp.float32),
        pltpu.VMEM((NB, EB), jnp.int32),
        pltpu.VMEM((NB, EB), jnp.int32),
        pltpu.VMEM((EB, CH), jnp.float32),
        pltpu.VMEM((EB, CH), jnp.float32),
        pltpu.VMEM((EB, CH), jnp.float32),
        pltpu.VMEM((RB, CH), jnp.float32),
        pltpu.VMEM((RB, CH), jnp.float32),
        pltpu.VMEM((RB, CH), jnp.float32),
        pltpu.SemaphoreType.DMA,
        pltpu.SemaphoreType.DMA,
        pltpu.SemaphoreType.DMA,
        pltpu.SemaphoreType.DMA,
        pltpu.SemaphoreType.DMA,
        pltpu.SemaphoreType.DMA,
    ],
)(_lp_body)


def kernel(x, edge_index, label, train_idx):
  n = x.shape[0]
  row = edge_index[0]
  col = edge_index[1]
  loop = jnp.arange(n, dtype=row.dtype)
  rowf = jnp.concatenate([row, loop])
  colf = jnp.concatenate([col, loop])

  deg = jnp.zeros((n,), jnp.float32).at[colf].add(1.0)
  dis = lax.rsqrt(deg)
  d2 = dis * dis

  oh = jax.nn.one_hot(label[train_idx, 0], C, dtype=jnp.float32)
  y = jnp.zeros((n, C), jnp.float32).at[train_idx].set(oh)
  yb = 0.1 * dis[:, None] * y

  def expand(v):
    if v.ndim == 1:
      v = jnp.broadcast_to(v[:, None], (n, 2 * CH))
    else:
      v = jnp.pad(v, ((0, 0), (0, 2 * CH - v.shape[1])))
    vp = jnp.zeros((NP, 2 * CH), jnp.float32).at[:n].set(v)
    return vp.reshape(NP, 2, CH).transpose(1, 0, 2)

  d2e = expand(d2)
  ybe = expand(yb)
  dinve = expand(jnp.sqrt(deg))

  pad = jnp.full((EPAD - (E + n),), NP - 1, dtype=jnp.int32)
  colt = jnp.concatenate([colf, pad]).reshape(NS, NB, EB)
  rowt = jnp.concatenate([rowf, pad]).reshape(NS, NB, EB)

  out, _ = _lp_call(colt, rowt, d2e, ybe, dinve)
  res = jnp.concatenate([out[0], out[1]], axis=1)
  return res[:n, :C]


# speedup vs baseline: 25.5475x; 1.0471x over previous
import functools

import jax
import jax.numpy as jnp
from jax import lax
from jax.experimental import pallas as pl
from jax.experimental.pallas import tpu as pltpu
from jax.experimental.pallas import tpu_sc as plsc

N = 10000
E = 320000
C = 40
ALPHA = 0.9
NUM_ITERS = 50

NS = 16
L = 16
CH = 32
NP = 10240
RT = NP // NS
RB = 128
NRC = RT // RB
EB = 384
NB = 54
EPAD = NS * NB * EB
NBUF = 3


def _lp_body(colt, rowt, d2e, ybe, dinve, out, s_hbm, a_sh,
             colv, rowv, gb0, gb1, gb2, da, dd, dy,
             gsem0, gsem1, gsem2, ssem0, ssem1, ssem2):
  gb = (gb0, gb1, gb2)
  gsem = (gsem0, gsem1, gsem2)
  ssem = (ssem0, ssem1, ssem2)
  sid = lax.axis_index("s")
  cid = lax.axis_index("c")
  base_row = sid * RT

  pltpu.sync_copy(colt.at[sid], colv)
  pltpu.sync_copy(rowt.at[sid], rowv)

  def dense(mode):
    @pl.loop(0, NRC)
    def _chunk(k):
      rb = base_row + k * RB
      if mode != 0:
        pltpu.sync_copy(a_sh.at[pl.ds(rb, RB)], da)
        pltpu.sync_copy(d2e.at[cid, pl.ds(rb, RB)], dd)
      if mode != 1:
        pltpu.sync_copy(ybe.at[cid, pl.ds(rb, RB)], dy)

      @pl.loop(0, RB)
      def _row(r):
        for j in range(CH // L):
          sl = pl.ds(j * L, L)
          if mode == 0:
            da[r, sl] = dy[r, sl] * 10.0
          elif mode == 1:
            da[r, sl] = da[r, sl] * dd[r, sl]
          else:
            da[r, sl] = da[r, sl] * dd[r, sl] * ALPHA + dy[r, sl]

      pltpu.sync_copy(da, s_hbm.at[cid, pl.ds(rb, RB)])

      @pl.loop(0, RB)
      def _zrow(r):
        for j in range(CH // L):
          da[r, pl.ds(j * L, L)] = jnp.zeros((L,), jnp.float32)

      pltpu.sync_copy(da, a_sh.at[pl.ds(rb, RB)])

  def edge_pass():
    def fire_g(b, k):
      pltpu.async_copy(s_hbm.at[cid].at[colv.at[b]], gb[k], gsem[k])

    def wait_g(k):
      pltpu.make_async_copy(s_hbm.at[cid].at[colv.at[0]], gb[k],
                            gsem[k]).wait()

    def fire_s(b, k):
      pltpu.async_copy(gb[k], a_sh.at[rowv.at[b]], ssem[k], add=True)

    def wait_s(k):
      pltpu.make_async_copy(gb[k], a_sh.at[rowv.at[0]], ssem[k]).wait()

    fire_g(0, 0)
    fire_g(1, 1)
    wait_g(0)
    fire_s(0, 0)
    fire_g(2, 2)
    wait_g(1)
    fire_s(1, 1)

    @pl.loop(0, (NB - NBUF) // NBUF)
    def _grp(i):
      b0 = 2 + i * NBUF
      for j in range(NBUF):
        b = b0 + j
        k = (2 + j) % NBUF
        kn = (k + 1) % NBUF
        wait_s(kn)
        fire_g(b + 1, kn)
        wait_g(k)
        fire_s(b, k)

    wait_g((NB - 1) % NBUF)
    fire_s(NB - 1, (NB - 1) % NBUF)
    for k in range(NBUF):
      wait_s(k)

  dense(0)
  plsc.subcore_barrier()

  @pl.loop(0, NUM_ITERS)
  def _iter(i):
    edge_pass()
    plsc.subcore_barrier()
    dense(1)
    plsc.subcore_barrier()
    edge_pass()
    plsc.subcore_barrier()
    dense(2)
    plsc.subcore_barrier()

  @pl.loop(0, NRC)
  def _chunk(k):
    rb = base_row + k * RB
    pltpu.sync_copy(s_hbm.at[cid, pl.ds(rb, RB)], da)
    pltpu.sync_copy(dinve.at[cid, pl.ds(rb, RB)], dd)

    @pl.loop(0, RB)
    def _row(r):
      for j in range(CH // L):
        sl = pl.ds(j * L, L)
        da[r, sl] = da[r, sl] * dd[r, sl]

    pltpu.sync_copy(da, out.at[cid, pl.ds(rb, RB)])


_lp_call = functools.partial(
    pl.kernel,
    out_type=(jax.ShapeDtypeStruct((2, NP, CH), jnp.float32),
              jax.ShapeDtypeStruct((2, NP, CH), jnp.float32)),
    mesh=plsc.VectorSubcoreMesh(core_axis_name="c", subcore_axis_name="s"),
    compiler_params=pltpu.CompilerParams(use_tc_tiling_on_sc=False),
    scratch_types=[
        pltpu.VMEM_SHARED((NP, CH), jnp.float32),
        pltpu.VMEM((NB, EB), jnp.int32),
        pltpu.VMEM((NB, EB), jnp.int32),
        pltpu.VMEM((EB, CH), jnp.float32),
        pltpu.VMEM((EB, CH), jnp.float32),
        pltpu.VMEM((EB, CH), jnp.float32),
        pltpu.VMEM((RB, CH), jnp.float32),
        pltpu.VMEM((RB, CH), jnp.float32),
        pltpu.VMEM((RB, CH), jnp.float32),
        pltpu.SemaphoreType.DMA,
        pltpu.SemaphoreType.DMA,
        pltpu.SemaphoreType.DMA,
        pltpu.SemaphoreType.DMA,
        pltpu.SemaphoreType.DMA,
        pltpu.SemaphoreType.DMA,
    ],
)(_lp_body)


def kernel(x, edge_index, label, train_idx):
  n = x.shape[0]
  row = edge_index[0]
  col = edge_index[1]
  loop = jnp.arange(n, dtype=row.dtype)
  rowf = jnp.concatenate([row, loop])
  colf = jnp.concatenate([col, loop])

  deg = jnp.zeros((n,), jnp.float32).at[colf].add(1.0)
  dis = lax.rsqrt(deg)
  d2 = dis * dis

  oh = jax.nn.one_hot(label[train_idx, 0], C, dtype=jnp.float32)
  y = jnp.zeros((n, C), jnp.float32).at[train_idx].set(oh)
  yb = 0.1 * dis[:, None] * y

  def expand(v):
    if v.ndim == 1:
      v = jnp.broadcast_to(v[:, None], (n, 2 * CH))
    else:
      v = jnp.pad(v, ((0, 0), (0, 2 * CH - v.shape[1])))
    vp = jnp.zeros((NP, 2 * CH), jnp.float32).at[:n].set(v)
    return vp.reshape(NP, 2, CH).transpose(1, 0, 2)

  d2e = expand(d2)
  ybe = expand(yb)
  dinve = expand(jnp.sqrt(deg))

  pad = jnp.full((EPAD - (E + n),), NP - 1, dtype=jnp.int32)
  colt = jnp.concatenate([colf, pad]).reshape(NS, NB, EB)
  rowt = jnp.concatenate([rowf, pad]).reshape(NS, NB, EB)

  out, _ = _lp_call(colt, rowt, d2e, ybe, dinve)
  res = jnp.concatenate([out[0], out[1]], axis=1)
  return res[:n, :C]


# speedup vs baseline: 26.3616x; 1.0319x over previous
import functools

import jax
import jax.numpy as jnp
from jax import lax
from jax.experimental import pallas as pl
from jax.experimental.pallas import tpu as pltpu
from jax.experimental.pallas import tpu_sc as plsc

N = 10000
E = 320000
C = 40
ALPHA = 0.9
NUM_ITERS = 50

NS = 16
L = 16
CH = 32
NP = 10240
RT = NP // NS
RB = 128
NRC = RT // RB
EB = 384
NB = 54
EPAD = NS * NB * EB
NBUF = 3


def _lp_body(colt, rowt, d2e, ybe, dinve, out, s_hbm, a_sh,
             colv, rowv, gb0, gb1, gb2, da, dd, dy,
             gsem0, gsem1, gsem2, ssem0, ssem1, ssem2):
  gb = (gb0, gb1, gb2)
  gsem = (gsem0, gsem1, gsem2)
  ssem = (ssem0, ssem1, ssem2)
  sid = lax.axis_index("s")
  cid = lax.axis_index("c")
  base_row = sid * RT

  pltpu.sync_copy(colt.at[sid], colv)
  pltpu.sync_copy(rowt.at[sid], rowv)

  def dense(mode):
    @pl.loop(0, NRC)
    def _chunk(k):
      rb = base_row + k * RB

      @pl.when(k > 0)
      def _wait_prev_a_write():
        pltpu.make_async_copy(da, a_sh.at[pl.ds(base_row, RB)],
                              ssem0).wait()

      if mode != 0:
        pltpu.async_copy(a_sh.at[pl.ds(rb, RB)], da, gsem0)
        pltpu.async_copy(d2e.at[cid, pl.ds(rb, RB)], dd, gsem1)
      if mode != 1:
        pltpu.async_copy(ybe.at[cid, pl.ds(rb, RB)], dy, gsem2)
      if mode != 0:
        pltpu.make_async_copy(a_sh.at[pl.ds(rb, RB)], da, gsem0).wait()
        pltpu.make_async_copy(d2e.at[cid, pl.ds(rb, RB)], dd, gsem1).wait()
      if mode != 1:
        pltpu.make_async_copy(ybe.at[cid, pl.ds(rb, RB)], dy, gsem2).wait()

      @pl.loop(0, RB)
      def _row(r):
        for j in range(CH // L):
          sl = pl.ds(j * L, L)
          if mode == 0:
            da[r, sl] = dy[r, sl] * 10.0
          elif mode == 1:
            da[r, sl] = da[r, sl] * dd[r, sl]
          else:
            da[r, sl] = da[r, sl] * dd[r, sl] * ALPHA + dy[r, sl]

      pltpu.sync_copy(da, s_hbm.at[cid, pl.ds(rb, RB)])

      @pl.loop(0, RB)
      def _zrow(r):
        for j in range(CH // L):
          da[r, pl.ds(j * L, L)] = jnp.zeros((L,), jnp.float32)

      pltpu.async_copy(da, a_sh.at[pl.ds(rb, RB)], ssem0)

    pltpu.make_async_copy(da, a_sh.at[pl.ds(base_row, RB)], ssem0).wait()

  def edge_pass():
    def fire_g(b, k):
      pltpu.async_copy(s_hbm.at[cid].at[colv.at[b]], gb[k], gsem[k])

    def wait_g(k):
      pltpu.make_async_copy(s_hbm.at[cid].at[colv.at[0]], gb[k],
                            gsem[k]).wait()

    def fire_s(b, k):
      pltpu.async_copy(gb[k], a_sh.at[rowv.at[b]], ssem[k], add=True)

    def wait_s(k):
      pltpu.make_async_copy(gb[k], a_sh.at[rowv.at[0]], ssem[k]).wait()

    fire_g(0, 0)
    fire_g(1, 1)
    wait_g(0)
    fire_s(0, 0)
    fire_g(2, 2)
    wait_g(1)
    fire_s(1, 1)

    @pl.loop(0, (NB - NBUF) // NBUF)
    def _grp(i):
      b0 = 2 + i * NBUF
      for j in range(NBUF):
        b = b0 + j
        k = (2 + j) % NBUF
        kn = (k + 1) % NBUF
        wait_s(kn)
        fire_g(b + 1, kn)
        wait_g(k)
        fire_s(b, k)

    wait_g((NB - 1) % NBUF)
    fire_s(NB - 1, (NB - 1) % NBUF)
    for k in range(NBUF):
      wait_s(k)

  dense(0)
  plsc.subcore_barrier()

  @pl.loop(0, NUM_ITERS)
  def _iter(i):
    edge_pass()
    plsc.subcore_barrier()
    dense(1)
    plsc.subcore_barrier()
    edge_pass()
    plsc.subcore_barrier()
    dense(2)
    plsc.subcore_barrier()

  @pl.loop(0, NRC)
  def _chunk(k):
    rb = base_row + k * RB
    pltpu.sync_copy(s_hbm.at[cid, pl.ds(rb, RB)], da)
    pltpu.sync_copy(dinve.at[cid, pl.ds(rb, RB)], dd)

    @pl.loop(0, RB)
    def _row(r):
      for j in range(CH // L):
        sl = pl.ds(j * L, L)
        da[r, sl] = da[r, sl] * dd[r, sl]

    pltpu.sync_copy(da, out.at[cid, pl.ds(rb, RB)])


_lp_call = functools.partial(
    pl.kernel,
    out_type=(jax.ShapeDtypeStruct((2, NP, CH), jnp.float32),
              jax.ShapeDtypeStruct((2, NP, CH), jnp.float32)),
    mesh=plsc.VectorSubcoreMesh(core_axis_name="c", subcore_axis_name="s"),
    compiler_params=pltpu.CompilerParams(use_tc_tiling_on_sc=False),
    scratch_types=[
        pltpu.VMEM_SHARED((NP, CH), jnp.float32),
        pltpu.VMEM((NB, EB), jnp.int32),
        pltpu.VMEM((NB, EB), jnp.int32),
        pltpu.VMEM((EB, CH), jnp.float32),
        pltpu.VMEM((EB, CH), jnp.float32),
        pltpu.VMEM((EB, CH), jnp.float32),
        pltpu.VMEM((RB, CH), jnp.float32),
        pltpu.VMEM((RB, CH), jnp.float32),
        pltpu.VMEM((RB, CH), jnp.float32),
        pltpu.SemaphoreType.DMA,
        pltpu.SemaphoreType.DMA,
        pltpu.SemaphoreType.DMA,
        pltpu.SemaphoreType.DMA,
        pltpu.SemaphoreType.DMA,
        pltpu.SemaphoreType.DMA,
    ],
)(_lp_body)


def kernel(x, edge_index, label, train_idx):
  n = x.shape[0]
  row = edge_index[0]
  col = edge_index[1]
  loop = jnp.arange(n, dtype=row.dtype)
  rowf = jnp.concatenate([row, loop])
  colf = jnp.concatenate([col, loop])

  deg = jnp.zeros((n,), jnp.float32).at[colf].add(1.0)
  dis = lax.rsqrt(deg)
  d2 = dis * dis

  oh = jax.nn.one_hot(label[train_idx, 0], C, dtype=jnp.float32)
  y = jnp.zeros((n, C), jnp.float32).at[train_idx].set(oh)
  yb = 0.1 * dis[:, None] * y

  def expand(v):
    if v.ndim == 1:
      v = jnp.broadcast_to(v[:, None], (n, 2 * CH))
    else:
      v = jnp.pad(v, ((0, 0), (0, 2 * CH - v.shape[1])))
    vp = jnp.zeros((NP, 2 * CH), jnp.float32).at[:n].set(v)
    return vp.reshape(NP, 2, CH).transpose(1, 0, 2)

  d2e = expand(d2)
  ybe = expand(yb)
  dinve = expand(jnp.sqrt(deg))

  pad = jnp.full((EPAD - (E + n),), NP - 1, dtype=jnp.int32)
  colt = jnp.concatenate([colf, pad]).reshape(NS, NB, EB)
  rowt = jnp.concatenate([rowf, pad]).reshape(NS, NB, EB)

  out, _ = _lp_call(colt, rowt, d2e, ybe, dinve)
  res = jnp.concatenate([out[0], out[1]], axis=1)
  return res[:n, :C]
